# Initial kernel scaffold; baseline (speedup 1.0000x reference)
import jax
import jax.numpy as jnp
from jax.experimental import pallas as pl


def kernel(x, edge_index, kernel_idx, W, gamma, beta):
    raise NotImplementedError("write your pallas kernel here")


# speedup vs baseline: 2.9513x; 2.9513x over previous
import functools

import jax
import jax.numpy as jnp
from jax import lax
from jax.experimental import pallas as pl
from jax.experimental.pallas import tpu as pltpu
from jax.experimental.pallas import tpu_sc as plsc

N = 10000
E = 320000
INC = 128
OUTC = 128
K = 27
EPS = 1e-5

NC = 2
NS = 16
NW = NC * NS
EPW = E // NW
CHUNK = 80
NCHUNK = EPW // CHUNK
ACC_N = 10240
ROWS_PER_TILE = ACC_N // NS
ZROWS = 128


def _xw_body(x_ref, w_ref, o_ref):
    o_ref[0] = jnp.dot(x_ref[...], w_ref[0], preferred_element_type=jnp.float32)


def _xw(x, W):
    return pl.pallas_call(
        _xw_body,
        grid=(K,),
        in_specs=[
            pl.BlockSpec((N, INC), lambda k: (0, 0)),
            pl.BlockSpec((1, INC, OUTC), lambda k: (k, 0, 0)),
        ],
        out_specs=pl.BlockSpec((1, N, OUTC), lambda k: (k, 0, 0)),
        out_shape=jax.ShapeDtypeStruct((K, N, OUTC), jnp.float32),
    )(x, W)


def _sc_body(table, gidx_h, dst_h, out_h, idx_v, dst_v, rows_v, zbuf, acc, sem):
    cid = lax.axis_index("c")
    sid = lax.axis_index("s")
    wid = sid * NC + cid

    def _zero_row(r, _):
        for c in range(OUTC // 16):
            zbuf[r, pl.ds(c * 16, 16)] = jnp.zeros((16,), jnp.float32)
        return 0

    lax.fori_loop(0, ZROWS, _zero_row, 0)
    for j in range(ROWS_PER_TILE // ZROWS):
        pltpu.sync_copy(zbuf, acc.at[pl.ds(sid * ROWS_PER_TILE + j * ZROWS, ZROWS), :])
    plsc.subcore_barrier()

    base = wid * EPW

    def _chunk(c, _):
        off = base + c * CHUNK
        pltpu.sync_copy(gidx_h.at[pl.ds(off, CHUNK)], idx_v)
        pltpu.sync_copy(dst_h.at[pl.ds(off, CHUNK)], dst_v)
        pltpu.async_copy(table.at[idx_v], rows_v, sem).wait()
        pltpu.sync_copy(rows_v, acc.at[dst_v], add=True)
        return 0

    lax.fori_loop(0, NCHUNK, _chunk, 0)
    plsc.subcore_barrier()

    for j in range(ROWS_PER_TILE // ZROWS):
        r0 = sid * ROWS_PER_TILE + j * ZROWS
        pltpu.sync_copy(acc.at[pl.ds(r0, ZROWS), :], out_h.at[cid, pl.ds(r0, ZROWS), :])


def _sc_scatter(table, gidx, dst):
    mesh = plsc.VectorSubcoreMesh(core_axis_name="c", subcore_axis_name="s")
    f = functools.partial(
        pl.kernel,
        mesh=mesh,
        out_type=jax.ShapeDtypeStruct((NC, ACC_N, OUTC), jnp.float32),
        scratch_types=[
            pltpu.VMEM((CHUNK,), jnp.int32),
            pltpu.VMEM((CHUNK,), jnp.int32),
            pltpu.VMEM((CHUNK, OUTC), jnp.float32),
            pltpu.VMEM((ZROWS, OUTC), jnp.float32),
            pltpu.VMEM_SHARED((ACC_N, OUTC), jnp.float32),
            pltpu.SemaphoreType.DMA,
        ],
    )(_sc_body)
    return f(table, gidx, dst)


def _bn_body(p_ref, g_ref, b_ref, o_ref):
    s = p_ref[0] + p_ref[1]
    mean = jnp.mean(s, axis=0, keepdims=True)
    d = s - mean
    var = jnp.mean(d * d, axis=0, keepdims=True)
    y = d * lax.rsqrt(var + EPS) * g_ref[...] + b_ref[...]
    o_ref[...] = jnp.maximum(y, 0.0)


def _bn(partials, gamma, beta):
    return pl.pallas_call(
        _bn_body,
        grid=(1,),
        in_specs=[
            pl.BlockSpec((NC, N, OUTC), lambda i: (0, 0, 0)),
            pl.BlockSpec((1, OUTC), lambda i: (0, 0)),
            pl.BlockSpec((1, OUTC), lambda i: (0, 0)),
        ],
        out_specs=pl.BlockSpec((N, OUTC), lambda i: (0, 0)),
        out_shape=jax.ShapeDtypeStruct((N, OUTC), jnp.float32),
    )(partials, gamma, beta)


def kernel(x, edge_index, kernel_idx, W, gamma, beta):
    src = edge_index[0]
    dst = edge_index[1]
    gidx = kernel_idx.astype(jnp.int32) * N + src
    xw = _xw(x, W)
    table = xw.reshape(K * N, OUTC)
    partials = _sc_scatter(table, gidx, dst)
    return _bn(partials, gamma.reshape(1, OUTC), beta.reshape(1, OUTC))


# speedup vs baseline: 4.3467x; 1.4728x over previous
import functools

import jax
import jax.numpy as jnp
from jax import lax
from jax.experimental import pallas as pl
from jax.experimental.pallas import tpu as pltpu
from jax.experimental.pallas import tpu_sc as plsc

N = 10000
E = 320000
INC = 128
OUTC = 128
K = 27
EPS = 1e-5

NC = 2
NS = 16
HALF = OUTC // NC
EPT = E // NS
CHUNK = 40
NCHUNK = EPT // CHUNK
NPASS = 4
NCHUNK_P = NCHUNK // NPASS
ACC_N = 10240
ROWS_PER_TILE = ACC_N // NS

GROUP = 5
NGROUP = NCHUNK_P // GROUP
NPAIR = (NGROUP - 2) // 2


def _xw_body(x_ref, w_ref, o_ref):
    o_ref[0] = jnp.dot(x_ref[...], w_ref[0], preferred_element_type=jnp.float32)


def _xw(x, W):
    return pl.pallas_call(
        _xw_body,
        grid=(K,),
        in_specs=[
            pl.BlockSpec((N, INC), lambda k: (0, 0)),
            pl.BlockSpec((1, INC, OUTC), lambda k: (k, 0, 0)),
        ],
        out_specs=pl.BlockSpec((1, N, OUTC), lambda k: (k, 0, 0)),
        out_shape=jax.ShapeDtypeStruct((K, N, OUTC), jnp.float32),
    )(x, W)


def _sc_body(table, gidx_h, dst_h, out_h, idx2, dst2, rows, acc, sem_g, sem_s):
    cid = lax.axis_index("c")
    sid = lax.axis_index("s")

    def _zero_row(r, _):
        for c in range(HALF // 16):
            rows[0, r, pl.ds(c * 16, 16)] = jnp.zeros((16,), jnp.float32)
        return 0

    lax.fori_loop(0, CHUNK, _zero_row, 0)
    for j in range(ROWS_PER_TILE // CHUNK):
        pltpu.sync_copy(
            rows.at[0], acc.at[pl.ds(sid * ROWS_PER_TILE + j * CHUNK, CHUNK), :]
        )
    plsc.subcore_barrier()

    def _fire_g(gi, bank):
        for b in range(GROUP):
            c = gi * GROUP + b
            pltpu.make_async_copy(table.at[idx2.at[c]], rows.at[bank * GROUP + b], sem_g).start()

    def _drain_g(gi, bank):
        for b in range(GROUP):
            c = gi * GROUP + b
            pltpu.make_async_copy(table.at[idx2.at[c]], rows.at[bank * GROUP + b], sem_g).wait()

    def _fire_s(gi, bank):
        for b in range(GROUP):
            c = gi * GROUP + b
            pltpu.async_copy(rows.at[bank * GROUP + b], acc.at[dst2.at[c]], sem_s, add=True)

    def _drain_s(gi, bank):
        for b in range(GROUP):
            c = gi * GROUP + b
            pltpu.make_async_copy(rows.at[bank * GROUP + b], acc.at[dst2.at[c]], sem_s).wait()

    def _pair(i, _):
        gi = 2 * i + 1
        _drain_g(gi, 1)
        _fire_s(gi, 1)
        _drain_s(gi - 1, 0)
        _fire_g(gi + 1, 0)
        gi2 = 2 * i + 2
        _drain_g(gi2, 0)
        _fire_s(gi2, 0)
        _drain_s(gi2 - 1, 1)
        _fire_g(gi2 + 1, 1)
        return 0

    for h in range(NPASS):
        pltpu.sync_copy(gidx_h.at[cid, sid, h], idx2)
        pltpu.sync_copy(dst_h.at[sid, h], dst2)

        _fire_g(0, 0)
        _drain_g(0, 0)
        _fire_s(0, 0)
        _fire_g(1, 1)

        lax.fori_loop(0, NPAIR, _pair, 0)

        for gi in range(2 * NPAIR + 1, NGROUP):
            p = gi % 2
            _drain_g(gi, p)
            _fire_s(gi, p)
            _drain_s(gi - 1, 1 - p)
            if gi + 1 < NGROUP:
                _fire_g(gi + 1, 1 - p)
        _drain_s(NGROUP - 1, (NGROUP - 1) % 2)

    plsc.subcore_barrier()

    for j in range(ROWS_PER_TILE // CHUNK):
        r0 = sid * ROWS_PER_TILE + j * CHUNK
        pltpu.sync_copy(acc.at[pl.ds(r0, CHUNK), :], out_h.at[cid, pl.ds(r0, CHUNK), :])


def _sc_scatter(table, gidx, dst):
    mesh = plsc.VectorSubcoreMesh(core_axis_name="c", subcore_axis_name="s")
    f = functools.partial(
        pl.kernel,
        mesh=mesh,
        compiler_params=pltpu.CompilerParams(use_tc_tiling_on_sc=False),
        out_type=jax.ShapeDtypeStruct((NC, ACC_N, HALF), jnp.float32),
        scratch_types=[
            pltpu.VMEM((NCHUNK_P, CHUNK), jnp.int32),
            pltpu.VMEM((NCHUNK_P, CHUNK), jnp.int32),
            pltpu.VMEM((2 * GROUP, CHUNK, HALF), jnp.float32),
            pltpu.VMEM_SHARED((ACC_N, HALF), jnp.float32),
            pltpu.SemaphoreType.DMA,
            pltpu.SemaphoreType.DMA,
        ],
    )(_sc_body)
    return f(table, gidx, dst)


def _bn_body(p_ref, g_ref, b_ref, o_ref):
    s = jnp.concatenate([p_ref[0], p_ref[1]], axis=1)
    mean = jnp.mean(s, axis=0, keepdims=True)
    d = s - mean
    var = jnp.mean(d * d, axis=0, keepdims=True)
    y = d * lax.rsqrt(var + EPS) * g_ref[...] + b_ref[...]
    o_ref[...] = jnp.maximum(y, 0.0)


def _bn(partials, gamma, beta):
    return pl.pallas_call(
        _bn_body,
        grid=(1,),
        in_specs=[
            pl.BlockSpec((NC, N, HALF), lambda i: (0, 0, 0)),
            pl.BlockSpec((1, OUTC), lambda i: (0, 0)),
            pl.BlockSpec((1, OUTC), lambda i: (0, 0)),
        ],
        out_specs=pl.BlockSpec((N, OUTC), lambda i: (0, 0)),
        out_shape=jax.ShapeDtypeStruct((N, OUTC), jnp.float32),
    )(partials, gamma, beta)


def kernel(x, edge_index, kernel_idx, W, gamma, beta):
    src = edge_index[0]
    dst = edge_index[1]
    gidx = kernel_idx.astype(jnp.int32) * N + src
    gidx2 = jnp.stack([2 * gidx, 2 * gidx + 1]).reshape(NC, NS, NPASS, NCHUNK_P, CHUNK)
    dst4 = dst.reshape(NS, NPASS, NCHUNK_P, CHUNK)
    xw = _xw(x, W)
    table = xw.reshape(K * N * NC, HALF)
    partials = _sc_scatter(table, gidx2, dst4)
    return _bn(partials, gamma.reshape(1, OUTC), beta.reshape(1, OUTC))


# speedup vs baseline: 5.2228x; 1.2016x over previous
import functools

import jax
import jax.numpy as jnp
from jax import lax
from jax.experimental import pallas as pl
from jax.experimental.pallas import tpu as pltpu
from jax.experimental.pallas import tpu_sc as plsc

N = 10000
E = 320000
INC = 128
OUTC = 128
K = 27
EPS = 1e-5

NC = 2
NS = 16
HALF = OUTC // NC
EPT = E // NS
CHUNK = 80
NCHUNK = EPT // CHUNK
NPASS = 2
NCHUNK_P = NCHUNK // NPASS
ACC_N = 10240
ROWS_PER_TILE = ACC_N // NS

GROUP = 5
NGROUP = NCHUNK_P // GROUP
NPAIR = (NGROUP - 2) // 2


def _xw_body(x_ref, w_ref, o_ref):
    o_ref[0] = jnp.dot(x_ref[...], w_ref[0], preferred_element_type=jnp.float32)


def _xw(x, W):
    return pl.pallas_call(
        _xw_body,
        grid=(K,),
        in_specs=[
            pl.BlockSpec((N, INC), lambda k: (0, 0)),
            pl.BlockSpec((1, INC, OUTC), lambda k: (k, 0, 0)),
        ],
        out_specs=pl.BlockSpec((1, N, OUTC), lambda k: (k, 0, 0)),
        out_shape=jax.ShapeDtypeStruct((K, N, OUTC), jnp.float32),
    )(x, W)


def _sc_body(table, gidx_h, dst_h, out_h, idx2, dst2, rows, acc, sem_g, sem_s):
    cid = lax.axis_index("c")
    sid = lax.axis_index("s")

    def _zero_row(r, _):
        for c in range(HALF // 16):
            rows[0, r, pl.ds(c * 16, 16)] = jnp.zeros((16,), jnp.float32)
        return 0

    lax.fori_loop(0, CHUNK, _zero_row, 0)
    for j in range(ROWS_PER_TILE // CHUNK):
        pltpu.sync_copy(
            rows.at[0], acc.at[pl.ds(sid * ROWS_PER_TILE + j * CHUNK, CHUNK), :]
        )
    plsc.subcore_barrier()

    def _fire_g(gi, bank):
        for b in range(GROUP):
            c = gi * GROUP + b
            pltpu.make_async_copy(table.at[idx2.at[c]], rows.at[bank * GROUP + b], sem_g).start()

    def _drain_g(gi, bank):
        for b in range(GROUP):
            c = gi * GROUP + b
            pltpu.make_async_copy(table.at[idx2.at[c]], rows.at[bank * GROUP + b], sem_g).wait()

    def _fire_s(gi, bank):
        for b in range(GROUP):
            c = gi * GROUP + b
            pltpu.async_copy(rows.at[bank * GROUP + b], acc.at[dst2.at[c]], sem_s, add=True)

    def _drain_s(gi, bank):
        for b in range(GROUP):
            c = gi * GROUP + b
            pltpu.make_async_copy(rows.at[bank * GROUP + b], acc.at[dst2.at[c]], sem_s).wait()

    def _pair(i, _):
        gi = 2 * i + 1
        _drain_g(gi, 1)
        _fire_s(gi, 1)
        _drain_s(gi - 1, 0)
        _fire_g(gi + 1, 0)
        gi2 = 2 * i + 2
        _drain_g(gi2, 0)
        _fire_s(gi2, 0)
        _drain_s(gi2 - 1, 1)
        _fire_g(gi2 + 1, 1)
        return 0

    for h in range(NPASS):
        pltpu.sync_copy(gidx_h.at[cid, sid, h], idx2)
        pltpu.sync_copy(dst_h.at[sid, h], dst2)

        _fire_g(0, 0)
        _drain_g(0, 0)
        _fire_s(0, 0)
        _fire_g(1, 1)

        lax.fori_loop(0, NPAIR, _pair, 0)

        for gi in range(2 * NPAIR + 1, NGROUP):
            p = gi % 2
            _drain_g(gi, p)
            _fire_s(gi, p)
            _drain_s(gi - 1, 1 - p)
            if gi + 1 < NGROUP:
                _fire_g(gi + 1, 1 - p)
        _drain_s(NGROUP - 1, (NGROUP - 1) % 2)

    plsc.subcore_barrier()

    for j in range(ROWS_PER_TILE // CHUNK):
        r0 = sid * ROWS_PER_TILE + j * CHUNK
        pltpu.sync_copy(acc.at[pl.ds(r0, CHUNK), :], out_h.at[cid, pl.ds(r0, CHUNK), :])


def _sc_scatter(table, gidx, dst):
    mesh = plsc.VectorSubcoreMesh(core_axis_name="c", subcore_axis_name="s")
    f = functools.partial(
        pl.kernel,
        mesh=mesh,
        compiler_params=pltpu.CompilerParams(use_tc_tiling_on_sc=False),
        out_type=jax.ShapeDtypeStruct((NC, ACC_N, HALF), jnp.float32),
        scratch_types=[
            pltpu.VMEM((NCHUNK_P, CHUNK), jnp.int32),
            pltpu.VMEM((NCHUNK_P, CHUNK), jnp.int32),
            pltpu.VMEM((2 * GROUP, CHUNK, HALF), jnp.float32),
            pltpu.VMEM_SHARED((ACC_N, HALF), jnp.float32),
            pltpu.SemaphoreType.DMA,
            pltpu.SemaphoreType.DMA,
        ],
    )(_sc_body)
    return f(table, gidx, dst)


def _bn_body(p_ref, g_ref, b_ref, o_ref):
    s = jnp.concatenate([p_ref[0], p_ref[1]], axis=1)
    mean = jnp.mean(s, axis=0, keepdims=True)
    d = s - mean
    var = jnp.mean(d * d, axis=0, keepdims=True)
    y = d * lax.rsqrt(var + EPS) * g_ref[...] + b_ref[...]
    o_ref[...] = jnp.maximum(y, 0.0)


def _bn(partials, gamma, beta):
    return pl.pallas_call(
        _bn_body,
        grid=(1,),
        in_specs=[
            pl.BlockSpec((NC, N, HALF), lambda i: (0, 0, 0)),
            pl.BlockSpec((1, OUTC), lambda i: (0, 0)),
            pl.BlockSpec((1, OUTC), lambda i: (0, 0)),
        ],
        out_specs=pl.BlockSpec((N, OUTC), lambda i: (0, 0)),
        out_shape=jax.ShapeDtypeStruct((N, OUTC), jnp.float32),
    )(partials, gamma, beta)


def kernel(x, edge_index, kernel_idx, W, gamma, beta):
    src = edge_index[0]
    dst = edge_index[1]
    gidx = kernel_idx.astype(jnp.int32) * N + src
    gidx2 = jnp.stack([2 * gidx, 2 * gidx + 1]).reshape(NC, NS, NPASS, NCHUNK_P, CHUNK)
    dst4 = dst.reshape(NS, NPASS, NCHUNK_P, CHUNK)
    xw = _xw(x, W)
    table = xw.reshape(K * N * NC, HALF)
    partials = _sc_scatter(table, gidx2, dst4)
    return _bn(partials, gamma.reshape(1, OUTC), beta.reshape(1, OUTC))
